# baseline (speedup 1.0000x reference)
import functools

import jax
import jax.numpy as jnp
from jax import lax
from jax.experimental import pallas as pl
from jax.experimental.pallas import tpu as pltpu
from jax.experimental.pallas import tpu_sc as plsc

N = 10000
E = 320000
D = 128

NC = 2
NS = 16
NW = NC * NS
EPW = E // NW
CH = 40
NCHUNK = EPW // CH
NPAD = 10240
RPT = NPAD // NS
DEGW = 16

_MESH = plsc.VectorSubcoreMesh(core_axis_name="c", subcore_axis_name="s")


@functools.partial(
    pl.kernel,
    out_type=[
        jax.ShapeDtypeStruct((NC, NPAD, D), jnp.float32),
        jax.ShapeDtypeStruct((NC, NPAD, DEGW), jnp.float32),
    ],
    mesh=_MESH,
    compiler_params=pltpu.CompilerParams(use_tc_tiling_on_sc=False),
    scratch_types=[
        [pltpu.VMEM((2, CH), jnp.int32) for _ in range(6)],
        [pltpu.VMEM((CH, D), jnp.float32) for _ in range(3)],
        pltpu.VMEM((CH, DEGW), jnp.float32),
        pltpu.VMEM((CH, DEGW), jnp.float32),
        pltpu.VMEM_SHARED((NPAD, D), jnp.float32),
        pltpu.VMEM_SHARED((NPAD, DEGW), jnp.float32),
        [pltpu.SemaphoreType.DMA for _ in range(6)],
        [pltpu.SemaphoreType.DMA for _ in range(3)],
        [pltpu.SemaphoreType.DMA for _ in range(3)],
    ],
)
def _sc_scatter(ei_hbm, h_hbm, zeros_agg_hbm, zeros_deg_hbm, ones_hbm,
                agg_out, deg_out,
                idx_r, rows_r, ones_v, zdeg_v, agg_sh, deg_sh,
                sem_i, sem_g, sem_s):
    cid = lax.axis_index("c")
    sid = lax.axis_index("s")
    wid = sid * NC + cid
    wbase = wid * EPW

    zrow = rows_r[0]
    pltpu.sync_copy(ones_hbm, ones_v)
    pltpu.sync_copy(zeros_agg_hbm, zrow)
    pltpu.sync_copy(zeros_deg_hbm, zdeg_v)
    row0 = sid * RPT

    @pl.loop(0, RPT // CH)
    def _zero(j):
        pltpu.sync_copy(zrow, agg_sh.at[pl.ds(row0 + j * CH, CH)])
        pltpu.sync_copy(zdeg_v, deg_sh.at[pl.ds(row0 + j * CH, CH)])

    plsc.subcore_barrier()

    def fetch_idx(c, k):
        base = wbase + c * CH
        pltpu.async_copy(ei_hbm.at[:, pl.ds(base, CH)], idx_r[k], sem_i[k])

    def wait_idx(k):
        pltpu.make_async_copy(ei_hbm.at[:, pl.ds(0, CH)], idx_r[k], sem_i[k]).wait()

    def fire_gather(k, b):
        pltpu.async_copy(h_hbm.at[idx_r[k].at[0]], rows_r[b], sem_g[b])

    def wait_gather(k, b):
        pltpu.make_async_copy(h_hbm.at[idx_r[k].at[0]], rows_r[b], sem_g[b]).wait()

    def fire_scatter(k, b):
        pltpu.async_copy(rows_r[b], agg_sh.at[idx_r[k].at[1]], sem_s[b], add=True)
        pltpu.async_copy(ones_v, deg_sh.at[idx_r[k].at[1]], sem_s[b], add=True)

    def wait_scatter(k, b):
        pltpu.make_async_copy(rows_r[b], agg_sh.at[idx_r[k].at[1]], sem_s[b]).wait()
        pltpu.make_async_copy(ones_v, deg_sh.at[idx_r[k].at[1]], sem_s[b]).wait()

    def step(c, ki, kr):
        wait_scatter((ki - 3) % 6, kr)
        wait_idx(ki)
        fire_gather(ki, kr)
        wait_gather((ki - 2) % 6, (kr - 2) % 3)
        fire_scatter((ki - 2) % 6, (kr - 2) % 3)
        fetch_idx(c + 3, (ki + 3) % 6)

    for c in (0, 1, 2):
        fetch_idx(c, c)
    for c in (0, 1, 2):
        wait_idx(c)
        fire_gather(c, c)
        if c == 2:
            wait_gather(0, 0)
            fire_scatter(0, 0)
        fetch_idx(c + 3, c + 3)

    NSTEADY = NCHUNK - 8
    @pl.loop(0, NSTEADY // 6)
    def _main(j):
        c0 = 3 + 6 * j
        for k in range(6):
            step(c0 + k, (3 + k) % 6, k % 3)

    for c in range(3 + 6 * (NSTEADY // 6), NCHUNK - 3):
        step(c, c % 6, c % 3)

    for c in range(NCHUNK - 3, NCHUNK):
        wait_scatter((c - 3) % 6, (c - 3) % 3)
        wait_idx(c % 6)
        fire_gather(c % 6, c % 3)
        wait_gather((c - 2) % 6, (c - 2) % 3)
        fire_scatter((c - 2) % 6, (c - 2) % 3)
    for c in range(NCHUNK - 2, NCHUNK):
        wait_gather(c % 6, c % 3)
        fire_scatter(c % 6, c % 3)
    for c in range(NCHUNK - 3, NCHUNK):
        wait_scatter(c % 6, c % 3)

    plsc.subcore_barrier()
    pltpu.sync_copy(agg_sh.at[pl.ds(row0, RPT)], agg_out.at[cid, pl.ds(row0, RPT)])
    pltpu.sync_copy(deg_sh.at[pl.ds(row0, RPT)], deg_out.at[cid, pl.ds(row0, RPT)])


_TC_R = 1000


def _tc_body(h_ref, agg_ref, deg_ref, ws_ref, wn_ref, b_ref, out_ref):
    agg = agg_ref[0] + agg_ref[1]
    deg2 = deg_ref[0] + deg_ref[1]
    deg = jnp.sum(deg2, axis=1, keepdims=True)
    mean = agg * (1.0 / jnp.maximum(deg, 1.0))
    acc = jnp.dot(h_ref[...], ws_ref[...], preferred_element_type=jnp.float32)
    acc = acc + jnp.dot(mean, wn_ref[...], preferred_element_type=jnp.float32)
    acc = acc + b_ref[...]
    out_ref[...] = h_ref[...] + jnp.maximum(acc, 0.0)


def _tc_dense(h, agg_p, deg_p, W_self, W_neigh, b2):
    grid = (N // _TC_R,)
    return pl.pallas_call(
        _tc_body,
        grid=grid,
        in_specs=[
            pl.BlockSpec((_TC_R, D), lambda i: (i, 0)),
            pl.BlockSpec((NC, _TC_R, D), lambda i: (0, i, 0)),
            pl.BlockSpec((NC, _TC_R, DEGW), lambda i: (0, i, 0)),
            pl.BlockSpec((D, D), lambda i: (0, 0)),
            pl.BlockSpec((D, D), lambda i: (0, 0)),
            pl.BlockSpec((1, D), lambda i: (0, 0)),
        ],
        out_specs=pl.BlockSpec((_TC_R, D), lambda i: (i, 0)),
        out_shape=jax.ShapeDtypeStruct((N, D), jnp.float32),
    )(h, agg_p, deg_p, W_self, W_neigh, b2)


def kernel(h, edge_index, W_self, W_neigh, b):
    ei = edge_index.astype(jnp.int32)
    zeros_agg = jnp.zeros((CH, D), jnp.float32)
    zeros_deg = jnp.zeros((CH, DEGW), jnp.float32)
    ones16 = jnp.zeros((CH, DEGW), jnp.float32).at[:, 0].set(1.0)
    agg_p, deg_p = _sc_scatter(ei, h, zeros_agg, zeros_deg, ones16)
    return _tc_dense(h, agg_p, deg_p, W_self, W_neigh, b.reshape(1, D))


# speedup vs baseline: 1.1542x; 1.1542x over previous
import functools

import jax
import jax.numpy as jnp
from jax import lax
from jax.experimental import pallas as pl
from jax.experimental.pallas import tpu as pltpu
from jax.experimental.pallas import tpu_sc as plsc

N = 10000
E = 320000
D = 128

NC = 2
NS = 16
NW = NC * NS
EPW = E // NW
CH = 80
NCHUNK = EPW // CH
NPAD = 10240
RPT = NPAD // NS
DEGW = 16

_MESH = plsc.VectorSubcoreMesh(core_axis_name="c", subcore_axis_name="s")


@functools.partial(
    pl.kernel,
    out_type=[
        jax.ShapeDtypeStruct((NC, NPAD, D), jnp.float32),
        jax.ShapeDtypeStruct((NC, NPAD, DEGW), jnp.float32),
    ],
    mesh=_MESH,
    compiler_params=pltpu.CompilerParams(use_tc_tiling_on_sc=False),
    scratch_types=[
        [pltpu.VMEM((2, CH), jnp.int32) for _ in range(6)],
        [pltpu.VMEM((CH, D), jnp.float32) for _ in range(3)],
        pltpu.VMEM((CH, DEGW), jnp.float32),
        pltpu.VMEM((CH, DEGW), jnp.float32),
        pltpu.VMEM_SHARED((NPAD, D), jnp.float32),
        pltpu.VMEM_SHARED((NPAD, DEGW), jnp.float32),
        [pltpu.SemaphoreType.DMA for _ in range(6)],
        [pltpu.SemaphoreType.DMA for _ in range(3)],
        [pltpu.SemaphoreType.DMA for _ in range(3)],
    ],
)
def _sc_scatter(ei_hbm, h_hbm, zeros_agg_hbm, zeros_deg_hbm, ones_hbm,
                agg_out, deg_out,
                idx_r, rows_r, ones_v, zdeg_v, agg_sh, deg_sh,
                sem_i, sem_g, sem_s):
    cid = lax.axis_index("c")
    sid = lax.axis_index("s")
    wid = sid * NC + cid
    wbase = wid * EPW

    zrow = rows_r[0]
    pltpu.sync_copy(ones_hbm, ones_v)
    pltpu.sync_copy(zeros_agg_hbm, zrow)
    pltpu.sync_copy(zeros_deg_hbm, zdeg_v)
    row0 = sid * RPT

    @pl.loop(0, RPT // CH)
    def _zero(j):
        pltpu.sync_copy(zrow, agg_sh.at[pl.ds(row0 + j * CH, CH)])
        pltpu.sync_copy(zdeg_v, deg_sh.at[pl.ds(row0 + j * CH, CH)])

    plsc.subcore_barrier()

    def fetch_idx(c, k):
        base = wbase + c * CH
        pltpu.async_copy(ei_hbm.at[:, pl.ds(base, CH)], idx_r[k], sem_i[k])

    def wait_idx(k):
        pltpu.make_async_copy(ei_hbm.at[:, pl.ds(0, CH)], idx_r[k], sem_i[k]).wait()

    def fire_gather(k, b):
        pltpu.async_copy(h_hbm.at[idx_r[k].at[0]], rows_r[b], sem_g[b])

    def wait_gather(k, b):
        pltpu.make_async_copy(h_hbm.at[idx_r[k].at[0]], rows_r[b], sem_g[b]).wait()

    def fire_scatter(k, b):
        pltpu.async_copy(rows_r[b], agg_sh.at[idx_r[k].at[1]], sem_s[b], add=True)
        pltpu.async_copy(ones_v, deg_sh.at[idx_r[k].at[1]], sem_s[b], add=True)

    def wait_scatter(k, b):
        pltpu.make_async_copy(rows_r[b], agg_sh.at[idx_r[k].at[1]], sem_s[b]).wait()
        pltpu.make_async_copy(ones_v, deg_sh.at[idx_r[k].at[1]], sem_s[b]).wait()

    def step(c, ki, kr):
        wait_scatter((ki - 3) % 6, kr)
        wait_idx(ki)
        fire_gather(ki, kr)
        wait_gather((ki - 2) % 6, (kr - 2) % 3)
        fire_scatter((ki - 2) % 6, (kr - 2) % 3)
        fetch_idx(c + 3, (ki + 3) % 6)

    for c in (0, 1, 2):
        fetch_idx(c, c)
    for c in (0, 1, 2):
        wait_idx(c)
        fire_gather(c, c)
        if c == 2:
            wait_gather(0, 0)
            fire_scatter(0, 0)
        fetch_idx(c + 3, c + 3)

    NSTEADY = NCHUNK - 8
    @pl.loop(0, NSTEADY // 6)
    def _main(j):
        c0 = 3 + 6 * j
        for k in range(6):
            step(c0 + k, (3 + k) % 6, k % 3)

    for c in range(3 + 6 * (NSTEADY // 6), NCHUNK - 3):
        step(c, c % 6, c % 3)

    for c in range(NCHUNK - 3, NCHUNK):
        wait_scatter((c - 3) % 6, (c - 3) % 3)
        wait_idx(c % 6)
        fire_gather(c % 6, c % 3)
        wait_gather((c - 2) % 6, (c - 2) % 3)
        fire_scatter((c - 2) % 6, (c - 2) % 3)
    for c in range(NCHUNK - 2, NCHUNK):
        wait_gather(c % 6, c % 3)
        fire_scatter(c % 6, c % 3)
    for c in range(NCHUNK - 3, NCHUNK):
        wait_scatter(c % 6, c % 3)

    plsc.subcore_barrier()
    pltpu.sync_copy(agg_sh.at[pl.ds(row0, RPT)], agg_out.at[cid, pl.ds(row0, RPT)])
    pltpu.sync_copy(deg_sh.at[pl.ds(row0, RPT)], deg_out.at[cid, pl.ds(row0, RPT)])


_TC_R = 2000


def _tc_body(h_ref, agg_ref, deg_ref, ws_ref, wn_ref, b_ref, out_ref):
    agg = agg_ref[0] + agg_ref[1]
    deg2 = deg_ref[0] + deg_ref[1]
    deg = jnp.sum(deg2, axis=1, keepdims=True)
    mean = agg * (1.0 / jnp.maximum(deg, 1.0))
    acc = jnp.dot(h_ref[...], ws_ref[...], preferred_element_type=jnp.float32)
    acc = acc + jnp.dot(mean, wn_ref[...], preferred_element_type=jnp.float32)
    acc = acc + b_ref[...]
    out_ref[...] = h_ref[...] + jnp.maximum(acc, 0.0)


def _tc_dense(h, agg_p, deg_p, W_self, W_neigh, b2):
    grid = (N // _TC_R,)
    return pl.pallas_call(
        _tc_body,
        grid=grid,
        in_specs=[
            pl.BlockSpec((_TC_R, D), lambda i: (i, 0)),
            pl.BlockSpec((NC, _TC_R, D), lambda i: (0, i, 0)),
            pl.BlockSpec((NC, _TC_R, DEGW), lambda i: (0, i, 0)),
            pl.BlockSpec((D, D), lambda i: (0, 0)),
            pl.BlockSpec((D, D), lambda i: (0, 0)),
            pl.BlockSpec((1, D), lambda i: (0, 0)),
        ],
        out_specs=pl.BlockSpec((_TC_R, D), lambda i: (i, 0)),
        out_shape=jax.ShapeDtypeStruct((N, D), jnp.float32),
    )(h, agg_p, deg_p, W_self, W_neigh, b2)


def kernel(h, edge_index, W_self, W_neigh, b):
    ei = edge_index.astype(jnp.int32)
    zeros_agg = jnp.zeros((CH, D), jnp.float32)
    zeros_deg = jnp.zeros((CH, DEGW), jnp.float32)
    ones16 = jnp.zeros((CH, DEGW), jnp.float32).at[:, 0].set(1.0)
    agg_p, deg_p = _sc_scatter(ei, h, zeros_agg, zeros_deg, ones16)
    return _tc_dense(h, agg_p, deg_p, W_self, W_neigh, b.reshape(1, D))


# speedup vs baseline: 1.1617x; 1.0065x over previous
import functools

import jax
import jax.numpy as jnp
from jax import lax
from jax.experimental import pallas as pl
from jax.experimental.pallas import tpu as pltpu
from jax.experimental.pallas import tpu_sc as plsc

N = 10000
E = 320000
D = 128

NC = 2
NS = 16
NW = NC * NS
EPW = E // NW
CH = 80
NCHUNK = EPW // CH
NPAD = 10240
RPT = NPAD // NS
DEGW = 16

_MESH = plsc.VectorSubcoreMesh(core_axis_name="c", subcore_axis_name="s")


@functools.partial(
    pl.kernel,
    out_type=[
        jax.ShapeDtypeStruct((NC, NPAD, D), jnp.float32),
        jax.ShapeDtypeStruct((NC, NPAD, DEGW), jnp.float32),
    ],
    mesh=_MESH,
    compiler_params=pltpu.CompilerParams(use_tc_tiling_on_sc=False),
    scratch_types=[
        [pltpu.VMEM((2, CH), jnp.int32) for _ in range(6)],
        [pltpu.VMEM((CH, D), jnp.float32) for _ in range(3)],
        pltpu.VMEM((CH, DEGW), jnp.float32),
        pltpu.VMEM((CH, DEGW), jnp.float32),
        pltpu.VMEM_SHARED((NPAD, D), jnp.float32),
        pltpu.VMEM_SHARED((NPAD, DEGW), jnp.float32),
        [pltpu.SemaphoreType.DMA for _ in range(6)],
        [pltpu.SemaphoreType.DMA for _ in range(3)],
        [pltpu.SemaphoreType.DMA for _ in range(3)],
    ],
)
def _sc_scatter(ei_hbm, h_hbm, zeros_agg_hbm, zeros_deg_hbm, ones_hbm,
                agg_out, deg_out,
                idx_r, rows_r, ones_v, zdeg_v, agg_sh, deg_sh,
                sem_i, sem_g, sem_s):
    cid = lax.axis_index("c")
    sid = lax.axis_index("s")
    wid = sid * NC + cid
    wbase = wid * EPW

    zrow = rows_r[0]
    pltpu.sync_copy(ones_hbm, ones_v)
    pltpu.sync_copy(zeros_agg_hbm, zrow)
    pltpu.sync_copy(zeros_deg_hbm, zdeg_v)
    row0 = sid * RPT

    for j in range(RPT // CH):
        pltpu.async_copy(zrow, agg_sh.at[pl.ds(row0 + j * CH, CH)], sem_g[0])
        pltpu.async_copy(zdeg_v, deg_sh.at[pl.ds(row0 + j * CH, CH)], sem_g[1])
    for j in range(RPT // CH):
        pltpu.make_async_copy(zrow, agg_sh.at[pl.ds(row0, CH)], sem_g[0]).wait()
        pltpu.make_async_copy(zdeg_v, deg_sh.at[pl.ds(row0, CH)], sem_g[1]).wait()

    plsc.subcore_barrier()

    def fetch_idx(c, k):
        base = wbase + c * CH
        pltpu.async_copy(ei_hbm.at[:, pl.ds(base, CH)], idx_r[k], sem_i[k])

    def wait_idx(k):
        pltpu.make_async_copy(ei_hbm.at[:, pl.ds(0, CH)], idx_r[k], sem_i[k]).wait()

    def fire_gather(k, b):
        pltpu.async_copy(h_hbm.at[idx_r[k].at[0]], rows_r[b], sem_g[b])

    def wait_gather(k, b):
        pltpu.make_async_copy(h_hbm.at[idx_r[k].at[0]], rows_r[b], sem_g[b]).wait()

    def fire_scatter(k, b):
        pltpu.async_copy(rows_r[b], agg_sh.at[idx_r[k].at[1]], sem_s[b], add=True)
        pltpu.async_copy(ones_v, deg_sh.at[idx_r[k].at[1]], sem_s[b], add=True)

    def wait_scatter(k, b):
        pltpu.make_async_copy(rows_r[b], agg_sh.at[idx_r[k].at[1]], sem_s[b]).wait()
        pltpu.make_async_copy(ones_v, deg_sh.at[idx_r[k].at[1]], sem_s[b]).wait()

    def step(c, ki, kr):
        wait_scatter((ki - 3) % 6, kr)
        wait_idx(ki)
        fire_gather(ki, kr)
        wait_gather((ki - 2) % 6, (kr - 2) % 3)
        fire_scatter((ki - 2) % 6, (kr - 2) % 3)
        fetch_idx(c + 3, (ki + 3) % 6)

    for c in (0, 1, 2):
        fetch_idx(c, c)
    for c in (0, 1, 2):
        wait_idx(c)
        fire_gather(c, c)
        if c == 2:
            wait_gather(0, 0)
            fire_scatter(0, 0)
        fetch_idx(c + 3, c + 3)

    NSTEADY = NCHUNK - 8
    @pl.loop(0, NSTEADY // 6)
    def _main(j):
        c0 = 3 + 6 * j
        for k in range(6):
            step(c0 + k, (3 + k) % 6, k % 3)

    for c in range(3 + 6 * (NSTEADY // 6), NCHUNK - 3):
        step(c, c % 6, c % 3)

    for c in range(NCHUNK - 3, NCHUNK):
        wait_scatter((c - 3) % 6, (c - 3) % 3)
        wait_idx(c % 6)
        fire_gather(c % 6, c % 3)
        wait_gather((c - 2) % 6, (c - 2) % 3)
        fire_scatter((c - 2) % 6, (c - 2) % 3)
    for c in range(NCHUNK - 2, NCHUNK):
        wait_gather(c % 6, c % 3)
        fire_scatter(c % 6, c % 3)
    for c in range(NCHUNK - 3, NCHUNK):
        wait_scatter(c % 6, c % 3)

    plsc.subcore_barrier()
    pltpu.async_copy(agg_sh.at[pl.ds(row0, RPT)], agg_out.at[cid, pl.ds(row0, RPT)], sem_g[0])
    pltpu.async_copy(deg_sh.at[pl.ds(row0, RPT)], deg_out.at[cid, pl.ds(row0, RPT)], sem_g[1])
    pltpu.make_async_copy(agg_sh.at[pl.ds(row0, RPT)], agg_out.at[cid, pl.ds(row0, RPT)], sem_g[0]).wait()
    pltpu.make_async_copy(deg_sh.at[pl.ds(row0, RPT)], deg_out.at[cid, pl.ds(row0, RPT)], sem_g[1]).wait()


_TC_R = 2000


def _tc_body(h_ref, agg_ref, deg_ref, ws_ref, wn_ref, b_ref, out_ref):
    agg = agg_ref[0] + agg_ref[1]
    deg2 = deg_ref[0] + deg_ref[1]
    deg = jnp.sum(deg2, axis=1, keepdims=True)
    mean = agg * (1.0 / jnp.maximum(deg, 1.0))
    acc = jnp.dot(h_ref[...], ws_ref[...], preferred_element_type=jnp.float32)
    acc = acc + jnp.dot(mean, wn_ref[...], preferred_element_type=jnp.float32)
    acc = acc + b_ref[...]
    out_ref[...] = h_ref[...] + jnp.maximum(acc, 0.0)


def _tc_dense(h, agg_p, deg_p, W_self, W_neigh, b2):
    grid = (N // _TC_R,)
    return pl.pallas_call(
        _tc_body,
        grid=grid,
        in_specs=[
            pl.BlockSpec((_TC_R, D), lambda i: (i, 0)),
            pl.BlockSpec((NC, _TC_R, D), lambda i: (0, i, 0)),
            pl.BlockSpec((NC, _TC_R, DEGW), lambda i: (0, i, 0)),
            pl.BlockSpec((D, D), lambda i: (0, 0)),
            pl.BlockSpec((D, D), lambda i: (0, 0)),
            pl.BlockSpec((1, D), lambda i: (0, 0)),
        ],
        out_specs=pl.BlockSpec((_TC_R, D), lambda i: (i, 0)),
        out_shape=jax.ShapeDtypeStruct((N, D), jnp.float32),
    )(h, agg_p, deg_p, W_self, W_neigh, b2)


def kernel(h, edge_index, W_self, W_neigh, b):
    ei = edge_index.astype(jnp.int32)
    zeros_agg = jnp.zeros((CH, D), jnp.float32)
    zeros_deg = jnp.zeros((CH, DEGW), jnp.float32)
    ones16 = jnp.zeros((CH, DEGW), jnp.float32).at[:, 0].set(1.0)
    agg_p, deg_p = _sc_scatter(ei, h, zeros_agg, zeros_deg, ones16)
    return _tc_dense(h, agg_p, deg_p, W_self, W_neigh, b.reshape(1, D))


# speedup vs baseline: 1.1618x; 1.0001x over previous
import functools

import jax
import jax.numpy as jnp
from jax import lax
from jax.experimental import pallas as pl
from jax.experimental.pallas import tpu as pltpu
from jax.experimental.pallas import tpu_sc as plsc

N = 10000
E = 320000
D = 128

NC = 2
NS = 16
NW = NC * NS
EPW = E // NW
CH = 80
NCHUNK = EPW // CH
NPAD = 10240
RPT = NPAD // NS
DEGW = 16

_MESH = plsc.VectorSubcoreMesh(core_axis_name="c", subcore_axis_name="s")


@functools.partial(
    pl.kernel,
    out_type=[
        jax.ShapeDtypeStruct((NC, NPAD, D), jnp.float32),
        jax.ShapeDtypeStruct((NC, NPAD, DEGW), jnp.float32),
    ],
    mesh=_MESH,
    compiler_params=pltpu.CompilerParams(use_tc_tiling_on_sc=False),
    scratch_types=[
        [pltpu.VMEM((2, CH), jnp.int32) for _ in range(6)],
        [pltpu.VMEM((CH, D), jnp.float32) for _ in range(3)],
        pltpu.VMEM((CH, DEGW), jnp.float32),
        pltpu.VMEM((CH, DEGW), jnp.float32),
        pltpu.VMEM_SHARED((NPAD, D), jnp.float32),
        pltpu.VMEM_SHARED((NPAD, DEGW), jnp.float32),
        [pltpu.SemaphoreType.DMA for _ in range(6)],
        [pltpu.SemaphoreType.DMA for _ in range(3)],
        [pltpu.SemaphoreType.DMA for _ in range(3)],
    ],
)
def _sc_scatter(ei_hbm, h_hbm, zeros_agg_hbm, zeros_deg_hbm, ones_hbm,
                agg_out, deg_out,
                idx_r, rows_r, ones_v, zdeg_v, agg_sh, deg_sh,
                sem_i, sem_g, sem_s):
    cid = lax.axis_index("c")
    sid = lax.axis_index("s")
    wid = sid * NC + cid
    wbase = wid * EPW

    zrow = rows_r[0]
    pltpu.sync_copy(ones_hbm, ones_v)
    pltpu.sync_copy(zeros_agg_hbm, zrow)
    pltpu.sync_copy(zeros_deg_hbm, zdeg_v)
    row0 = sid * RPT

    for j in range(RPT // CH):
        pltpu.async_copy(zrow, agg_sh.at[pl.ds(row0 + j * CH, CH)], sem_g[0])
        pltpu.async_copy(zdeg_v, deg_sh.at[pl.ds(row0 + j * CH, CH)], sem_g[1])
    for j in range(RPT // CH):
        pltpu.make_async_copy(zrow, agg_sh.at[pl.ds(row0, CH)], sem_g[0]).wait()
        pltpu.make_async_copy(zdeg_v, deg_sh.at[pl.ds(row0, CH)], sem_g[1]).wait()

    plsc.subcore_barrier()

    def fetch_idx(c, k):
        base = wbase + c * CH
        pltpu.async_copy(ei_hbm.at[:, pl.ds(base, CH)], idx_r[k], sem_i[k])

    def wait_idx(k):
        pltpu.make_async_copy(ei_hbm.at[:, pl.ds(0, CH)], idx_r[k], sem_i[k]).wait()

    def fire_gather(k, b):
        pltpu.async_copy(h_hbm.at[idx_r[k].at[0]], rows_r[b], sem_g[b])

    def wait_gather(k, b):
        pltpu.make_async_copy(h_hbm.at[idx_r[k].at[0]], rows_r[b], sem_g[b]).wait()

    def fire_scatter(k, b):
        pltpu.async_copy(rows_r[b], agg_sh.at[idx_r[k].at[1]], sem_s[b], add=True)
        pltpu.async_copy(ones_v, deg_sh.at[idx_r[k].at[1]], sem_s[b], add=True)

    def wait_scatter(k, b):
        pltpu.make_async_copy(rows_r[b], agg_sh.at[idx_r[k].at[1]], sem_s[b]).wait()
        pltpu.make_async_copy(ones_v, deg_sh.at[idx_r[k].at[1]], sem_s[b]).wait()

    def step(c, ki, kr):
        wait_scatter((ki - 3) % 6, kr)
        wait_idx(ki)
        fire_gather(ki, kr)
        wait_gather((ki - 2) % 6, (kr - 2) % 3)
        fire_scatter((ki - 2) % 6, (kr - 2) % 3)
        fetch_idx(c + 3, (ki + 3) % 6)

    for c in (0, 1, 2):
        fetch_idx(c, c)
    for c in (0, 1, 2):
        wait_idx(c)
        fire_gather(c, c)
        if c == 2:
            wait_gather(0, 0)
            fire_scatter(0, 0)
        fetch_idx(c + 3, c + 3)

    NSTEADY = NCHUNK - 8
    @pl.loop(0, NSTEADY // 6)
    def _main(j):
        c0 = 3 + 6 * j
        for k in range(6):
            step(c0 + k, (3 + k) % 6, k % 3)

    for c in range(3 + 6 * (NSTEADY // 6), NCHUNK - 3):
        step(c, c % 6, c % 3)

    for c in range(NCHUNK - 3, NCHUNK):
        wait_scatter((c - 3) % 6, (c - 3) % 3)
        wait_idx(c % 6)
        fire_gather(c % 6, c % 3)
        wait_gather((c - 2) % 6, (c - 2) % 3)
        fire_scatter((c - 2) % 6, (c - 2) % 3)
    for c in range(NCHUNK - 2, NCHUNK):
        wait_gather(c % 6, c % 3)
        fire_scatter(c % 6, c % 3)
    for c in range(NCHUNK - 3, NCHUNK):
        wait_scatter(c % 6, c % 3)

    plsc.subcore_barrier()
    pltpu.async_copy(agg_sh.at[pl.ds(row0, RPT)], agg_out.at[cid, pl.ds(row0, RPT)], sem_g[0])
    pltpu.async_copy(deg_sh.at[pl.ds(row0, RPT)], deg_out.at[cid, pl.ds(row0, RPT)], sem_g[1])
    pltpu.make_async_copy(agg_sh.at[pl.ds(row0, RPT)], agg_out.at[cid, pl.ds(row0, RPT)], sem_g[0]).wait()
    pltpu.make_async_copy(deg_sh.at[pl.ds(row0, RPT)], deg_out.at[cid, pl.ds(row0, RPT)], sem_g[1]).wait()


_TC_R = 5000


def _tc_body(h_ref, agg_ref, deg_ref, ws_ref, wn_ref, b_ref, out_ref):
    agg = agg_ref[0] + agg_ref[1]
    deg2 = deg_ref[0] + deg_ref[1]
    deg = jnp.sum(deg2, axis=1, keepdims=True)
    mean = agg * (1.0 / jnp.maximum(deg, 1.0))
    acc = jnp.dot(h_ref[...], ws_ref[...], preferred_element_type=jnp.float32)
    acc = acc + jnp.dot(mean, wn_ref[...], preferred_element_type=jnp.float32)
    acc = acc + b_ref[...]
    out_ref[...] = h_ref[...] + jnp.maximum(acc, 0.0)


def _tc_dense(h, agg_p, deg_p, W_self, W_neigh, b2):
    grid = (N // _TC_R,)
    return pl.pallas_call(
        _tc_body,
        grid=grid,
        in_specs=[
            pl.BlockSpec((_TC_R, D), lambda i: (i, 0)),
            pl.BlockSpec((NC, _TC_R, D), lambda i: (0, i, 0)),
            pl.BlockSpec((NC, _TC_R, DEGW), lambda i: (0, i, 0)),
            pl.BlockSpec((D, D), lambda i: (0, 0)),
            pl.BlockSpec((D, D), lambda i: (0, 0)),
            pl.BlockSpec((1, D), lambda i: (0, 0)),
        ],
        out_specs=pl.BlockSpec((_TC_R, D), lambda i: (i, 0)),
        out_shape=jax.ShapeDtypeStruct((N, D), jnp.float32),
    )(h, agg_p, deg_p, W_self, W_neigh, b2)


def kernel(h, edge_index, W_self, W_neigh, b):
    ei = edge_index.astype(jnp.int32)
    zeros_agg = jnp.zeros((CH, D), jnp.float32)
    zeros_deg = jnp.zeros((CH, DEGW), jnp.float32)
    ones16 = jnp.zeros((CH, DEGW), jnp.float32).at[:, 0].set(1.0)
    agg_p, deg_p = _sc_scatter(ei, h, zeros_agg, zeros_deg, ones16)
    return _tc_dense(h, agg_p, deg_p, W_self, W_neigh, b.reshape(1, D))
